# baseline (speedup 1.0000x reference)
import functools
import jax
import jax.numpy as jnp
from jax import lax
from jax.experimental import pallas as pl
from jax.experimental.pallas import tpu as pltpu
from jax.experimental.pallas import tpu_sc as plsc

B = 16384
H = 128
NC = 2
NS = 16
L = 16
NW = NC * NS
BPW = B // NW
C = 128
NCHUNK = BPW // C
GROUPS = C // L
NBUF = 3

_mesh = plsc.VectorSubcoreMesh(core_axis_name="c", subcore_axis_name="s")

_GATHER_DNUMS = lax.GatherDimensionNumbers(
    offset_dims=(), collapsed_slice_dims=(0,), start_index_map=(0,))


def _shfl(v, idx):
    return lax.gather(v, idx[:, None], _GATHER_DNUMS, (1,),
                      mode=lax.GatherScatterMode.PROMISE_IN_BOUNDS)


@functools.partial(
    pl.kernel,
    mesh=_mesh,
    out_type=jax.ShapeDtypeStruct((B,), jnp.float32),
    scratch_types=[
        pltpu.VMEM((BPW,), jnp.int32),
        pltpu.VMEM((BPW,), jnp.int32),
        pltpu.VMEM((NBUF, C, H), jnp.float32),
        pltpu.VMEM((NBUF, C, H), jnp.float32),
        pltpu.VMEM((BPW,), jnp.float32),
        pltpu.VMEM((BPW,), jnp.float32),
        pltpu.VMEM((BPW,), jnp.float32),
        pltpu.SemaphoreType.DMA,
        pltpu.SemaphoreType.DMA,
        pltpu.SemaphoreType.DMA,
        pltpu.SemaphoreType.DMA,
    ],
)
def _mf_sc(users_hbm, movies_hbm, ue_hbm, me_hbm, ub_hbm, mb_hbm, out_hbm,
           uidx_v, midx_v, urows_v, mrows_v, ub_v, mb_v,
           out_v, sem, sem0, sem1, sem2):
    wid = lax.axis_index("s") * NC + lax.axis_index("c")
    base = wid * BPW

    pltpu.sync_copy(users_hbm.at[pl.ds(base, BPW)], uidx_v)
    pltpu.sync_copy(movies_hbm.at[pl.ds(base, BPW)], midx_v)

    ub_dma = pltpu.async_copy(ub_hbm.at[uidx_v], ub_v, sem)
    mb_dma = pltpu.async_copy(mb_hbm.at[midx_v], mb_v, sem)

    slot_sems = [sem0, sem1, sem2]

    def start_chunk(k):
        s = k % NBUF
        return (
            pltpu.async_copy(ue_hbm.at[uidx_v.at[pl.ds(k * C, C)]],
                             urows_v.at[s], slot_sems[s]),
            pltpu.async_copy(me_hbm.at[midx_v.at[pl.ds(k * C, C)]],
                             mrows_v.at[s], slot_sems[s]),
        )

    pending = {0: start_chunk(0), 1: start_chunk(1)}
    ub_dma.wait()
    mb_dma.wait()

    for k in range(NCHUNK):
        if k + NBUF - 1 < NCHUNK:
            pending[k + NBUF - 1] = start_chunk(k + NBUF - 1)
        du, dm = pending.pop(k)
        du.wait()
        dm.wait()
        ur = urows_v.at[k % NBUF]
        mr = mrows_v.at[k % NBUF]

        lane = lax.iota(jnp.int32, L)

        def group_body(g, _):
            vecs = []
            for r in range(L):
                row = g * L + r
                ps = [ur[row, pl.ds(j * L, L)] * mr[row, pl.ds(j * L, L)]
                      for j in range(H // L)]
                while len(ps) > 1:
                    ps = [a + b for a, b in zip(ps[0::2], ps[1::2])]
                vecs.append(ps[0])
            for o in (1, 2, 4, 8):
                m = (lane & o) != 0
                idx = lane ^ o
                vecs = [jnp.where(m, b, a) + _shfl(jnp.where(m, a, b), idx)
                        for a, b in zip(vecs[0::2], vecs[1::2])]
            sl = pl.ds(k * C + g * L, L)
            out_v[sl] = vecs[0] + ub_v[sl] + mb_v[sl] + 1.0
            return 0

        lax.fori_loop(0, GROUPS, group_body, 0, unroll=2)

    pltpu.sync_copy(out_v, out_hbm.at[pl.ds(base, BPW)])


def kernel(users, movies, user_embedding, movie_embedding, user_bias, movie_bias):
    return _mf_sc(users.astype(jnp.int32), movies.astype(jnp.int32),
                  user_embedding, movie_embedding,
                  user_bias.reshape(-1), movie_bias.reshape(-1))


# speedup vs baseline: 1.0852x; 1.0852x over previous
import functools
import jax
import jax.numpy as jnp
from jax import lax
from jax.experimental import pallas as pl
from jax.experimental.pallas import tpu as pltpu
from jax.experimental.pallas import tpu_sc as plsc

B = 16384
H = 128
NC = 2
NS = 16
L = 16
NW = NC * NS
BPW = B // NW
C = 128
NCHUNK = BPW // C
GROUPS = C // L
NBUF = 3

_mesh = plsc.VectorSubcoreMesh(core_axis_name="c", subcore_axis_name="s")

_GATHER_DNUMS = lax.GatherDimensionNumbers(
    offset_dims=(), collapsed_slice_dims=(0,), start_index_map=(0,))


def _shfl(v, idx):
    return lax.gather(v, idx[:, None], _GATHER_DNUMS, (1,),
                      mode=lax.GatherScatterMode.PROMISE_IN_BOUNDS)


@functools.partial(
    pl.kernel,
    mesh=_mesh,
    out_type=jax.ShapeDtypeStruct((B,), jnp.float32),
    scratch_types=[
        pltpu.VMEM((BPW,), jnp.int32),
        pltpu.VMEM((BPW,), jnp.int32),
        pltpu.VMEM((NBUF, C, H), jnp.float32),
        pltpu.VMEM((NBUF, C, H), jnp.float32),
        pltpu.VMEM((BPW,), jnp.float32),
        pltpu.VMEM((BPW,), jnp.float32),
        pltpu.VMEM((BPW,), jnp.float32),
        pltpu.SemaphoreType.DMA,
        pltpu.SemaphoreType.DMA,
        pltpu.SemaphoreType.DMA,
        pltpu.SemaphoreType.DMA,
    ],
)
def _mf_sc(users_hbm, movies_hbm, ue_hbm, me_hbm, ub_hbm, mb_hbm, out_hbm,
           uidx_v, midx_v, urows_v, mrows_v, ub_v, mb_v,
           out_v, sem, sem0, sem1, sem2):
    wid = lax.axis_index("s") * NC + lax.axis_index("c")
    base = wid * BPW

    pltpu.sync_copy(users_hbm.at[pl.ds(base, BPW)], uidx_v)
    pltpu.sync_copy(movies_hbm.at[pl.ds(base, BPW)], midx_v)

    ub_dma = pltpu.async_copy(ub_hbm.at[uidx_v], ub_v, sem)
    mb_dma = pltpu.async_copy(mb_hbm.at[midx_v], mb_v, sem)

    slot_sems = [sem0, sem1, sem2]

    def start_chunk(k):
        s = k % NBUF
        return (
            pltpu.async_copy(ue_hbm.at[uidx_v.at[pl.ds(k * C, C)]],
                             urows_v.at[s], slot_sems[s]),
            pltpu.async_copy(me_hbm.at[midx_v.at[pl.ds(k * C, C)]],
                             mrows_v.at[s], slot_sems[s]),
        )

    pending = {0: start_chunk(0), 1: start_chunk(1)}
    ub_dma.wait()
    mb_dma.wait()

    for k in range(NCHUNK):
        if k + NBUF - 1 < NCHUNK:
            pending[k + NBUF - 1] = start_chunk(k + NBUF - 1)
        du, dm = pending.pop(k)
        du.wait()
        dm.wait()
        ur = urows_v.at[k % NBUF]
        mr = mrows_v.at[k % NBUF]

        lane = lax.iota(jnp.int32, L)

        def group_body(g, _):
            vecs = []
            for r in range(L):
                row = g * L + r
                ps = [ur[row, pl.ds(j * L, L)] * mr[row, pl.ds(j * L, L)]
                      for j in range(H // L)]
                while len(ps) > 1:
                    ps = [a + b for a, b in zip(ps[0::2], ps[1::2])]
                vecs.append(ps[0])
            for o in (1, 2, 4, 8):
                m = (lane & o) != 0
                idx = lane ^ o
                vecs = [jnp.where(m, b, a) + _shfl(jnp.where(m, a, b), idx)
                        for a, b in zip(vecs[0::2], vecs[1::2])]
            sl = pl.ds(k * C + g * L, L)
            out_v[sl] = vecs[0] + ub_v[sl] + mb_v[sl] + 1.0
            return 0

        lax.fori_loop(0, GROUPS, group_body, 0, unroll=False)

    pltpu.sync_copy(out_v, out_hbm.at[pl.ds(base, BPW)])


def kernel(users, movies, user_embedding, movie_embedding, user_bias, movie_bias):
    return _mf_sc(users.astype(jnp.int32), movies.astype(jnp.int32),
                  user_embedding, movie_embedding,
                  user_bias.reshape(-1), movie_bias.reshape(-1))


# speedup vs baseline: 1.1791x; 1.0866x over previous
import functools
import jax
import jax.numpy as jnp
from jax import lax
from jax.experimental import pallas as pl
from jax.experimental.pallas import tpu as pltpu
from jax.experimental.pallas import tpu_sc as plsc

B = 16384
H = 128
NC = 2
NS = 16
L = 16
NW = NC * NS
BPW = B // NW
C = 128
NCHUNK = BPW // C
GROUPS = C // L
NBUF = 3

_mesh = plsc.VectorSubcoreMesh(core_axis_name="c", subcore_axis_name="s")

_GATHER_DNUMS = lax.GatherDimensionNumbers(
    offset_dims=(), collapsed_slice_dims=(0,), start_index_map=(0,))


def _shfl(v, idx):
    return lax.gather(v, idx[:, None], _GATHER_DNUMS, (1,),
                      mode=lax.GatherScatterMode.PROMISE_IN_BOUNDS)


@functools.partial(
    pl.kernel,
    mesh=_mesh,
    out_type=jax.ShapeDtypeStruct((B,), jnp.float32),
    scratch_types=[
        pltpu.VMEM((BPW,), jnp.int32),
        pltpu.VMEM((BPW,), jnp.int32),
        pltpu.VMEM((NBUF, C, H), jnp.float32),
        pltpu.VMEM((NBUF, C, H), jnp.float32),
        pltpu.VMEM((BPW,), jnp.float32),
        pltpu.VMEM((BPW,), jnp.float32),
        pltpu.VMEM((BPW,), jnp.float32),
        pltpu.SemaphoreType.DMA,
        pltpu.SemaphoreType.DMA,
        pltpu.SemaphoreType.DMA,
        pltpu.SemaphoreType.DMA,
    ],
)
def _mf_sc(users_hbm, movies_hbm, ue_hbm, me_hbm, ub_hbm, mb_hbm, out_hbm,
           uidx_v, midx_v, urows_v, mrows_v, ub_v, mb_v,
           out_v, sem, sem0, sem1, sem2):
    wid = lax.axis_index("s") * NC + lax.axis_index("c")
    base = wid * BPW

    pltpu.sync_copy(users_hbm.at[pl.ds(base, BPW)], uidx_v)
    pltpu.sync_copy(movies_hbm.at[pl.ds(base, BPW)], midx_v)

    ub_dma = pltpu.async_copy(ub_hbm.at[uidx_v], ub_v, sem)
    mb_dma = pltpu.async_copy(mb_hbm.at[midx_v], mb_v, sem)

    slot_sems = [sem0, sem1, sem2]

    def start_chunk(k):
        s = k % NBUF
        return (
            pltpu.async_copy(ue_hbm.at[uidx_v.at[pl.ds(k * C, C)]],
                             urows_v.at[s], slot_sems[s]),
            pltpu.async_copy(me_hbm.at[midx_v.at[pl.ds(k * C, C)]],
                             mrows_v.at[s], slot_sems[s]),
        )

    pending = {0: start_chunk(0), 1: start_chunk(1)}
    ub_dma.wait()
    mb_dma.wait()

    for k in range(NCHUNK):
        if k + NBUF - 1 < NCHUNK:
            pending[k + NBUF - 1] = start_chunk(k + NBUF - 1)
        du, dm = pending.pop(k)
        du.wait()
        dm.wait()
        ur = urows_v.at[k % NBUF]
        mr = mrows_v.at[k % NBUF]

        lane = lax.iota(jnp.int32, L)

        def merge(a, b, o):
            m = (lane & o) != 0
            return jnp.where(m, b, a) + _shfl(jnp.where(m, a, b), lane ^ o)

        def group_body(g, _):
            stages = [None] * 4
            dots = None
            for r in range(L):
                row = g * L + r
                acc = ur[row, pl.ds(0, L)] * mr[row, pl.ds(0, L)]
                for j in range(1, H // L):
                    acc = acc + (ur[row, pl.ds(j * L, L)]
                                 * mr[row, pl.ds(j * L, L)])
                v = acc
                for lvl in range(4):
                    if stages[lvl] is None:
                        stages[lvl] = v
                        break
                    v = merge(stages[lvl], v, 1 << lvl)
                    stages[lvl] = None
                else:
                    dots = v
            sl = pl.ds(k * C + g * L, L)
            out_v[sl] = dots + ub_v[sl] + mb_v[sl] + 1.0
            return 0

        lax.fori_loop(0, GROUPS, group_body, 0, unroll=False)

    pltpu.sync_copy(out_v, out_hbm.at[pl.ds(base, BPW)])


def kernel(users, movies, user_embedding, movie_embedding, user_bias, movie_bias):
    return _mf_sc(users.astype(jnp.int32), movies.astype(jnp.int32),
                  user_embedding, movie_embedding,
                  user_bias.reshape(-1), movie_bias.reshape(-1))
